# baseline (speedup 1.0000x reference)
import functools

import jax
import jax.numpy as jnp
from jax import lax
from jax.experimental import pallas as pl
from jax.experimental.pallas import tpu as pltpu
from jax.experimental.pallas import tpu_sc as plsc

D_MODEL = 768
DPART = 256
S_TOTAL = 4096
NW = 32
S_PER_W = S_TOTAL // NW
CH = 16
N_CH = S_PER_W // CH


def _sc_fused(x, ix, iy, iz, Wx, Wy, Wz):
    B = x.shape[0]
    mesh = plsc.VectorSubcoreMesh(core_axis_name="c", subcore_axis_name="s")
    idx_t = pltpu.VMEM((S_PER_W,), jnp.int32)
    row_t = pltpu.VMEM((S_PER_W, DPART), jnp.float32)

    @functools.partial(
        pl.kernel,
        out_type=jax.ShapeDtypeStruct(x.shape, x.dtype),
        mesh=mesh,
        scratch_types=[
            idx_t, idx_t, idx_t, row_t, row_t, row_t,
            pltpu.VMEM((CH, D_MODEL), jnp.float32),
            pltpu.SemaphoreType.DMA, pltpu.SemaphoreType.DMA,
        ],
    )
    def k(x_hbm, ix_hbm, iy_hbm, iz_hbm, wx_hbm, wy_hbm, wz_hbm, o_hbm,
          ixv, iyv, izv, ex, ey, ez, xbuf, sem_i, sem_g):
        wid = lax.axis_index("s") * 2 + lax.axis_index("c")
        base = wid * S_PER_W
        sl = pl.ds(base, S_PER_W)
        ci = [pltpu.async_copy(h.at[sl], v, sem_i)
              for h, v in ((ix_hbm, ixv), (iy_hbm, iyv), (iz_hbm, izv))]
        for c in ci:
            c.wait()
        cg = [pltpu.async_copy(w.at[v], r, sem_g)
              for w, v, r in ((wx_hbm, ixv, ex), (wy_hbm, iyv, ey),
                              (wz_hbm, izv, ez))]
        for c in cg:
            c.wait()

        for b in range(B):
            def chunk_body(c, carry, b=b):
                r0 = c * CH
                rows = pl.ds(base + r0, CH)
                pltpu.sync_copy(x_hbm.at[b, rows], xbuf)

                def row_body(r, carry2):
                    for p, buf in enumerate((ex, ey, ez)):
                        for j in range(DPART // 16):
                            dst = pl.ds(p * DPART + j * 16, 16)
                            src = pl.ds(j * 16, 16)
                            xbuf[r, dst] = xbuf[r, dst] + buf[r0 + r, src]
                    return carry2

                lax.fori_loop(0, CH, row_body, 0)
                pltpu.sync_copy(xbuf, o_hbm.at[b, rows])
                return carry

            lax.fori_loop(0, N_CH, chunk_body, 0)

    return k(x, ix, iy, iz, Wx, Wy, Wz)


def kernel(x, src_tgt, src_pos_x, src_pos_y, src_pos_z, Wx, Wy, Wz):
    del src_tgt
    return _sc_fused(x, src_pos_x, src_pos_y, src_pos_z, Wx, Wy, Wz)


# speedup vs baseline: 7.2103x; 7.2103x over previous
import jax
import jax.numpy as jnp
from jax import lax
from jax.experimental import pallas as pl

D_MODEL = 768
DPART = 256
S_TOTAL = 4096
S_BLK = 4096
N_SBLK = S_TOTAL // S_BLK


def _body(ix_ref, iy_ref, iz_ref, x_ref, wx_ref, wy_ref, wz_ref, o_ref):
    iota = lax.broadcasted_iota(jnp.int32, (32, S_BLK), 0)

    def part(idx_ref, w_ref):
        oh = (idx_ref[0, 0, :][None, :] == iota).astype(jnp.float32)
        return lax.dot_general(
            oh, w_ref[...], (((0,), (0,)), ((), ())),
            preferred_element_type=jnp.float32,
        )

    ex = part(ix_ref, wx_ref)
    ey = part(iy_ref, wy_ref)
    ez = part(iz_ref, wz_ref)
    xb = x_ref[0]
    o_ref[0, :, 0:DPART] = xb[:, 0:DPART] + ex
    o_ref[0, :, DPART:2 * DPART] = xb[:, DPART:2 * DPART] + ey
    o_ref[0, :, 2 * DPART:D_MODEL] = xb[:, 2 * DPART:D_MODEL] + ez


def kernel(x, src_tgt, src_pos_x, src_pos_y, src_pos_z, Wx, Wy, Wz):
    del src_tgt
    B = x.shape[0]
    ix = src_pos_x.reshape(N_SBLK, 1, S_BLK)
    iy = src_pos_y.reshape(N_SBLK, 1, S_BLK)
    iz = src_pos_z.reshape(N_SBLK, 1, S_BLK)

    idx_spec = pl.BlockSpec((1, 1, S_BLK), lambda i, j: (i, 0, 0))
    tab_spec = pl.BlockSpec((32, DPART), lambda i, j: (0, 0))
    x_spec = pl.BlockSpec((1, S_BLK, D_MODEL), lambda i, j: (j, i, 0))

    return pl.pallas_call(
        _body,
        grid=(N_SBLK, B),
        in_specs=[idx_spec, idx_spec, idx_spec, x_spec, tab_spec, tab_spec,
                  tab_spec],
        out_specs=x_spec,
        out_shape=jax.ShapeDtypeStruct(x.shape, x.dtype),
    )(ix, iy, iz, x, Wx, Wy, Wz)
